# baseline (speedup 1.0000x reference)
import functools

import jax
import jax.numpy as jnp
from jax import lax
from jax.experimental import pallas as pl
from jax.experimental.pallas import tpu as pltpu
from jax.experimental.pallas import tpu_sc as plsc

N = 10000
E = 320000
DIN = 128
H = 128
R = 64
EPS = 1e-5

NC = 2
NS = 16
NW = NC * NS
LANES = 16
CH = 80
EPW = E // NW
NCHUNK = EPW // CH
NPAD = 10112
RPT = NPAD // NS
ZR = 8
FV = H // LANES

_MESH = plsc.VectorSubcoreMesh(
    core_axis_name="c", subcore_axis_name="s", num_cores=NC, num_subcores=NS
)


def _leaky(x):
    return jnp.where(x > 0, x, 0.01 * x)


def _deg_body(srcs3, dsts3, deg_out, deg_sp, si_v, di_v, rows_a, rows_b):
    cid = lax.axis_index("c")
    sid = lax.axis_index("s")
    wid = cid * NS + sid

    @pl.loop(0, CH)
    def _fill_zero(r):
        for j in range(FV):
            rows_b[r, pl.ds(j * LANES, LANES)] = jnp.zeros((LANES,), jnp.float32)

    for t in range(RPT // CH):
        r0 = sid * RPT + t * CH
        pltpu.sync_copy(rows_b, deg_sp.at[pl.ds(r0, CH)])
    _tail = RPT - (RPT // CH) * CH
    if _tail:
        pltpu.sync_copy(rows_b.at[pl.ds(0, _tail)],
                        deg_sp.at[pl.ds(sid * RPT + (RPT // CH) * CH, _tail)])

    @pl.loop(0, CH)
    def _fill_ones(r):
        for j in range(FV // 2):
            rows_a[r, pl.ds(j * LANES, LANES)] = jnp.ones((LANES,), jnp.float32)
            rows_b[r, pl.ds((FV // 2 + j) * LANES, LANES)] = jnp.ones(
                (LANES,), jnp.float32)
        for j in range(FV // 2):
            rows_a[r, pl.ds((FV // 2 + j) * LANES, LANES)] = jnp.zeros(
                (LANES,), jnp.float32)
    plsc.subcore_barrier()

    @pl.loop(0, NCHUNK)
    def _chunk(k):
        pltpu.sync_copy(srcs3.at[wid, k], si_v)
        pltpu.sync_copy(rows_a, deg_sp.at[si_v], add=True)
        pltpu.sync_copy(dsts3.at[wid, k], di_v)
        pltpu.sync_copy(rows_b, deg_sp.at[di_v], add=True)

    plsc.subcore_barrier()
    r0 = sid * RPT
    pltpu.sync_copy(deg_sp.at[pl.ds(r0, RPT)], deg_out.at[cid, pl.ds(r0, RPT)])


_deg_kernel = pl.kernel(
    _deg_body,
    out_type=jax.ShapeDtypeStruct((NC, NPAD, H), jnp.float32),
    mesh=_MESH,
    scratch_types=[
        pltpu.VMEM_SHARED((NPAD, H), jnp.float32),
        pltpu.VMEM((CH,), jnp.int32),
        pltpu.VMEM((CH,), jnp.int32),
        pltpu.VMEM((CH, H), jnp.float32),
        pltpu.VMEM((CH, H), jnp.float32),
    ],
)


def _scatter_body(hs, srcs3, dsts3, ewb3, out, agg_sp,
                  src_all, dst0, dst1, ew0, ew1, rows0, rows1,
                  sem0, sem1, gsem0, gsem1):
    cid = lax.axis_index("c")
    sid = lax.axis_index("s")
    wid = cid * NS + sid

    @pl.loop(0, CH)
    def _fill_zero(r):
        for j in range(FV):
            rows0[r, pl.ds(j * LANES, LANES)] = jnp.zeros((LANES,), jnp.float32)

    for t in range(RPT // CH):
        r0 = sid * RPT + t * CH
        pltpu.sync_copy(rows0, agg_sp.at[pl.ds(r0, CH)])
    _tail = RPT - (RPT // CH) * CH
    if _tail:
        pltpu.sync_copy(rows0.at[pl.ds(0, _tail)],
                        agg_sp.at[pl.ds(sid * RPT + (RPT // CH) * CH, _tail)])
    plsc.subcore_barrier()

    @pl.loop(0, NCHUNK)
    def _chunk(k):
        pltpu.sync_copy(srcs3.at[wid, k], src_all)
        pltpu.sync_copy(dsts3.at[wid, k], dst0)
        pltpu.sync_copy(ewb3.at[wid, k], ew0)
        pltpu.async_copy(hs.at[src_all], rows0, gsem0).wait()

        @pl.loop(0, CH)
        def _scale(e):
            w = ew0[e, :]
            for j in range(FV):
                sl = pl.ds(j * LANES, LANES)
                rows0[e, sl] = rows0[e, sl] * w

        pltpu.sync_copy(rows0, agg_sp.at[dst0], add=True)

    plsc.subcore_barrier()
    r0 = sid * RPT
    pltpu.sync_copy(agg_sp.at[pl.ds(r0, RPT)], out.at[cid, pl.ds(r0, RPT)])


_scatter_kernel = pl.kernel(
    _scatter_body,
    out_type=jax.ShapeDtypeStruct((NC, NPAD, H), jnp.float32),
    mesh=_MESH,
    scratch_types=[
        pltpu.VMEM_SHARED((NPAD, H), jnp.float32),
        pltpu.VMEM((CH,), jnp.int32),
        pltpu.VMEM((CH,), jnp.int32),
        pltpu.VMEM((CH,), jnp.int32),
        pltpu.VMEM((CH, LANES), jnp.float32),
        pltpu.VMEM((CH, LANES), jnp.float32),
        pltpu.VMEM((CH, H), jnp.float32),
        pltpu.VMEM((CH, H), jnp.float32),
        pltpu.SemaphoreType.DMA,
        pltpu.SemaphoreType.DMA,
        pltpu.SemaphoreType.DMA,
        pltpu.SemaphoreType.DMA,
    ],
)


def _inv_sqrt_deg(degs, half):
    d = (degs[0] + degs[1])[:N, half * (H // 2):half * (H // 2) + (H // 2)]
    d = jnp.max(d, axis=-1, keepdims=True)
    return jnp.where(d > 0, lax.rsqrt(d), 0.0)


def _tc_pre_body(x_ref, w1_ref, deg_ref, hs_ref, ro0_ref):
    x = x_ref[:]
    no = _inv_sqrt_deg(deg_ref[:], 0)
    hs_ref[:] = jnp.dot(x, w1_ref[:], preferred_element_type=jnp.float32) * no
    ro0_ref[:] = _leaky(jnp.mean(x, axis=0, keepdims=True))


_tc_pre = pl.pallas_call(
    _tc_pre_body,
    out_shape=[
        jax.ShapeDtypeStruct((N, H), jnp.float32),
        jax.ShapeDtypeStruct((1, DIN), jnp.float32),
    ],
)


def _tc_mid_body(agg_ref, deg_ref, a_ref, g_ref, b_ref,
                 pw_ref, pb_ref, rw_ref, rb_ref, wn_ref,
                 hsn_ref, ro_ref, mh_ref):
    ni = _inv_sqrt_deg(deg_ref[:], 1)
    y = (agg_ref[0] + agg_ref[1])[:N] * ni
    mu = jnp.mean(y, axis=0, keepdims=True)
    sub = y - a_ref[:] * mu
    var = jnp.mean(sub * sub, axis=0, keepdims=True)
    h = _leaky(g_ref[:] * sub * lax.rsqrt(var + EPS) + b_ref[:])
    phi = _leaky(jnp.dot(h, pw_ref[:], preferred_element_type=jnp.float32)
                 + pb_ref[:])
    ro = _leaky(jnp.dot(jnp.mean(phi, axis=0, keepdims=True), rw_ref[:],
                        preferred_element_type=jnp.float32) + rb_ref[:])
    ro_ref[:] = _leaky(ro)
    mh_ref[:] = _leaky(jnp.mean(h, axis=0, keepdims=True))
    no = _inv_sqrt_deg(deg_ref[:], 0)
    hsn_ref[:] = jnp.dot(h, wn_ref[:], preferred_element_type=jnp.float32) * no


_tc_mid = pl.pallas_call(
    _tc_mid_body,
    out_shape=[
        jax.ShapeDtypeStruct((N, H), jnp.float32),
        jax.ShapeDtypeStruct((1, R), jnp.float32),
        jax.ShapeDtypeStruct((1, H), jnp.float32),
    ],
)


def _tc_last_body(agg_ref, deg_ref, a_ref, g_ref, b_ref,
                  pw_ref, pb_ref, rw_ref, rb_ref,
                  ro_ref, mh_ref):
    ni = _inv_sqrt_deg(deg_ref[:], 1)
    y = (agg_ref[0] + agg_ref[1])[:N] * ni
    mu = jnp.mean(y, axis=0, keepdims=True)
    sub = y - a_ref[:] * mu
    var = jnp.mean(sub * sub, axis=0, keepdims=True)
    h = _leaky(g_ref[:] * sub * lax.rsqrt(var + EPS) + b_ref[:])
    phi = _leaky(jnp.dot(h, pw_ref[:], preferred_element_type=jnp.float32)
                 + pb_ref[:])
    ro = _leaky(jnp.dot(jnp.mean(phi, axis=0, keepdims=True), rw_ref[:],
                        preferred_element_type=jnp.float32) + rb_ref[:])
    ro_ref[:] = _leaky(ro)
    mh_ref[:] = _leaky(jnp.mean(h, axis=0, keepdims=True))


_tc_last = pl.pallas_call(
    _tc_last_body,
    out_shape=[
        jax.ShapeDtypeStruct((1, R), jnp.float32),
        jax.ShapeDtypeStruct((1, H), jnp.float32),
    ],
)


def kernel(node_feats, edge_index, edge_weights, W1, W2, W3,
           a1, g1, b1, a2, g2, b2, a3, g3, b3,
           p1W, p1b, r1W, r1b, p2W, p2b, r2W, r2b, p3W, p3b, r3W, r3b):
    srcs = edge_index[0]
    dsts = edge_index[1]
    row = lambda v: v.reshape(1, -1)

    srcs3 = srcs.reshape(NW, NCHUNK, CH)
    srcs2 = srcs.reshape(NW, EPW)
    dsts3 = dsts.reshape(NW, NCHUNK, CH)
    ewb3 = jnp.broadcast_to(edge_weights[:, None], (E, LANES)).reshape(
        NW, NCHUNK, CH, LANES)

    deg = _deg_kernel(srcs3, dsts3)

    hs1, ro0 = _tc_pre(node_feats, W1, deg)
    agg1 = _scatter_kernel(hs1, srcs3, dsts3, ewb3)
    hs2, ro1, mh1 = _tc_mid(agg1, deg, row(a1), row(g1), row(b1),
                            p1W, row(p1b), r1W, row(r1b), W2)
    agg2 = _scatter_kernel(hs2, srcs3, dsts3, ewb3)
    hs3, ro2, mh2 = _tc_mid(agg2, deg, row(a2), row(g2), row(b2),
                            p2W, row(p2b), r2W, row(r2b), W3)
    agg3 = _scatter_kernel(hs3, srcs3, dsts3, ewb3)
    ro3, mh3 = _tc_last(agg3, deg, row(a3), row(g3), row(b3),
                        p3W, row(p3b), r3W, row(r3b))

    return jnp.concatenate([ro0, ro1, mh1, ro2, mh2, ro3, mh3], axis=1)


# speedup vs baseline: 2.2376x; 2.2376x over previous
import functools

import jax
import jax.numpy as jnp
from jax import lax
from jax.experimental import pallas as pl
from jax.experimental.pallas import tpu as pltpu
from jax.experimental.pallas import tpu_sc as plsc

N = 10000
E = 320000
DIN = 128
H = 128
R = 64
EPS = 1e-5

NC = 2
NS = 16
NW = NC * NS
LANES = 16
CH = 80
EPW = E // NW
NCHUNK = EPW // CH
NPAD = 10112
RPT = NPAD // NS
ZR = 8
FV = H // LANES

_MESH = plsc.VectorSubcoreMesh(
    core_axis_name="c", subcore_axis_name="s", num_cores=NC, num_subcores=NS
)


def _leaky(x):
    return jnp.where(x > 0, x, 0.01 * x)


def _deg_body(srcs3, dsts3, deg_out, deg_sp, si_v, di_v, rows_a, rows_b):
    cid = lax.axis_index("c")
    sid = lax.axis_index("s")
    wid = cid * NS + sid

    @pl.loop(0, CH)
    def _fill_zero(r):
        for j in range(FV):
            rows_b[r, pl.ds(j * LANES, LANES)] = jnp.zeros((LANES,), jnp.float32)

    for t in range(RPT // CH):
        r0 = sid * RPT + t * CH
        pltpu.sync_copy(rows_b, deg_sp.at[pl.ds(r0, CH)])
    _tail = RPT - (RPT // CH) * CH
    if _tail:
        pltpu.sync_copy(rows_b.at[pl.ds(0, _tail)],
                        deg_sp.at[pl.ds(sid * RPT + (RPT // CH) * CH, _tail)])

    @pl.loop(0, CH)
    def _fill_ones(r):
        for j in range(FV // 2):
            rows_a[r, pl.ds(j * LANES, LANES)] = jnp.ones((LANES,), jnp.float32)
            rows_b[r, pl.ds((FV // 2 + j) * LANES, LANES)] = jnp.ones(
                (LANES,), jnp.float32)
        for j in range(FV // 2):
            rows_a[r, pl.ds((FV // 2 + j) * LANES, LANES)] = jnp.zeros(
                (LANES,), jnp.float32)
    plsc.subcore_barrier()

    @pl.loop(0, NCHUNK)
    def _chunk(k):
        pltpu.sync_copy(srcs3.at[wid, k], si_v)
        pltpu.sync_copy(rows_a, deg_sp.at[si_v], add=True)
        pltpu.sync_copy(dsts3.at[wid, k], di_v)
        pltpu.sync_copy(rows_b, deg_sp.at[di_v], add=True)

    plsc.subcore_barrier()
    r0 = sid * RPT
    pltpu.sync_copy(deg_sp.at[pl.ds(r0, RPT)], deg_out.at[cid, pl.ds(r0, RPT)])


_deg_kernel = pl.kernel(
    _deg_body,
    out_type=jax.ShapeDtypeStruct((NC, NPAD, H), jnp.float32),
    mesh=_MESH,
    scratch_types=[
        pltpu.VMEM_SHARED((NPAD, H), jnp.float32),
        pltpu.VMEM((CH,), jnp.int32),
        pltpu.VMEM((CH,), jnp.int32),
        pltpu.VMEM((CH, H), jnp.float32),
        pltpu.VMEM((CH, H), jnp.float32),
    ],
)


def _scatter_body(hs, srcs2, dsts3, ewb3, out, agg_sp,
                  src_all, dst0, dst1, ew0, ew1, rows0, rows1,
                  sem0, sem1, gsem0, gsem1):
    cid = lax.axis_index("c")
    sid = lax.axis_index("s")
    wid = cid * NS + sid

    @pl.loop(0, CH)
    def _fill_zero(r):
        for j in range(FV):
            rows0[r, pl.ds(j * LANES, LANES)] = jnp.zeros((LANES,), jnp.float32)

    for t in range(RPT // CH):
        r0 = sid * RPT + t * CH
        pltpu.sync_copy(rows0, agg_sp.at[pl.ds(r0, CH)])
    _tail = RPT - (RPT // CH) * CH
    if _tail:
        pltpu.sync_copy(rows0.at[pl.ds(0, _tail)],
                        agg_sp.at[pl.ds(sid * RPT + (RPT // CH) * CH, _tail)])
    pltpu.sync_copy(srcs2.at[wid], src_all)
    plsc.subcore_barrier()

    def start(k, dst_v, ew_v, rows_v, sem, gsem):
        pltpu.async_copy(dsts3.at[wid, k], dst_v, sem)
        pltpu.async_copy(ewb3.at[wid, k], ew_v, sem)
        pltpu.async_copy(hs.at[src_all.at[pl.ds(k * CH, CH)]], rows_v, gsem)

    def finish(k, dst_v, ew_v, rows_v, sem, gsem):
        pltpu.make_async_copy(dsts3.at[0, 0], dst_v, sem).wait()
        pltpu.make_async_copy(ewb3.at[0, 0], ew_v, sem).wait()
        pltpu.make_async_copy(hs.at[src_all.at[pl.ds(k * CH, CH)]],
                              rows_v, gsem).wait()

    def process(dst_v, ew_v, rows_v):
        @pl.loop(0, CH)
        def _scale(e):
            w = ew_v[pl.ds(e * LANES, LANES)]
            for j in range(FV):
                sl = pl.ds(j * LANES, LANES)
                rows_v[e, sl] = rows_v[e, sl] * w

        pltpu.sync_copy(rows_v, agg_sp.at[dst_v], add=True)

    start(0, dst0, ew0, rows0, sem0, gsem0)
    start(1, dst1, ew1, rows1, sem1, gsem1)

    @pl.loop(0, (NCHUNK + 1) // 2)
    def _pair(g):
        k0 = 2 * g
        finish(k0, dst0, ew0, rows0, sem0, gsem0)
        process(dst0, ew0, rows0)

        @pl.when(k0 + 2 < NCHUNK)
        def _():
            start(k0 + 2, dst0, ew0, rows0, sem0, gsem0)

        @pl.when(k0 + 1 < NCHUNK)
        def _():
            finish(k0 + 1, dst1, ew1, rows1, sem1, gsem1)
            process(dst1, ew1, rows1)

            @pl.when(k0 + 3 < NCHUNK)
            def _():
                start(k0 + 3, dst1, ew1, rows1, sem1, gsem1)

    plsc.subcore_barrier()
    r0 = sid * RPT
    pltpu.sync_copy(agg_sp.at[pl.ds(r0, RPT)], out.at[cid, pl.ds(r0, RPT)])


_scatter_kernel = pl.kernel(
    _scatter_body,
    out_type=jax.ShapeDtypeStruct((NC, NPAD, H), jnp.float32),
    mesh=_MESH,
    scratch_types=[
        pltpu.VMEM_SHARED((NPAD, H), jnp.float32),
        pltpu.VMEM((EPW,), jnp.int32),
        pltpu.VMEM((CH,), jnp.int32),
        pltpu.VMEM((CH,), jnp.int32),
        pltpu.VMEM((CH * LANES,), jnp.float32),
        pltpu.VMEM((CH * LANES,), jnp.float32),
        pltpu.VMEM((CH, H), jnp.float32),
        pltpu.VMEM((CH, H), jnp.float32),
        pltpu.SemaphoreType.DMA,
        pltpu.SemaphoreType.DMA,
        pltpu.SemaphoreType.DMA,
        pltpu.SemaphoreType.DMA,
    ],
)


def _inv_sqrt_deg(degs, half):
    d = (degs[0] + degs[1])[:N, half * (H // 2):half * (H // 2) + (H // 2)]
    d = jnp.max(d, axis=-1, keepdims=True)
    return jnp.where(d > 0, lax.rsqrt(d), 0.0)


def _tc_pre_body(x_ref, w1_ref, deg_ref, hs_ref, ro0_ref):
    x = x_ref[:]
    no = _inv_sqrt_deg(deg_ref[:], 0)
    hs_ref[:] = jnp.dot(x, w1_ref[:], preferred_element_type=jnp.float32) * no
    ro0_ref[:] = _leaky(jnp.mean(x, axis=0, keepdims=True))


_tc_pre = pl.pallas_call(
    _tc_pre_body,
    out_shape=[
        jax.ShapeDtypeStruct((N, H), jnp.float32),
        jax.ShapeDtypeStruct((1, DIN), jnp.float32),
    ],
)


def _tc_mid_body(agg_ref, deg_ref, a_ref, g_ref, b_ref,
                 pw_ref, pb_ref, rw_ref, rb_ref, wn_ref,
                 hsn_ref, ro_ref, mh_ref):
    ni = _inv_sqrt_deg(deg_ref[:], 1)
    y = (agg_ref[0] + agg_ref[1])[:N] * ni
    mu = jnp.mean(y, axis=0, keepdims=True)
    sub = y - a_ref[:] * mu
    var = jnp.mean(sub * sub, axis=0, keepdims=True)
    h = _leaky(g_ref[:] * sub * lax.rsqrt(var + EPS) + b_ref[:])
    phi = _leaky(jnp.dot(h, pw_ref[:], preferred_element_type=jnp.float32)
                 + pb_ref[:])
    ro = _leaky(jnp.dot(jnp.mean(phi, axis=0, keepdims=True), rw_ref[:],
                        preferred_element_type=jnp.float32) + rb_ref[:])
    ro_ref[:] = _leaky(ro)
    mh_ref[:] = _leaky(jnp.mean(h, axis=0, keepdims=True))
    no = _inv_sqrt_deg(deg_ref[:], 0)
    hsn_ref[:] = jnp.dot(h, wn_ref[:], preferred_element_type=jnp.float32) * no


_tc_mid = pl.pallas_call(
    _tc_mid_body,
    out_shape=[
        jax.ShapeDtypeStruct((N, H), jnp.float32),
        jax.ShapeDtypeStruct((1, R), jnp.float32),
        jax.ShapeDtypeStruct((1, H), jnp.float32),
    ],
)


def _tc_last_body(agg_ref, deg_ref, a_ref, g_ref, b_ref,
                  pw_ref, pb_ref, rw_ref, rb_ref,
                  ro_ref, mh_ref):
    ni = _inv_sqrt_deg(deg_ref[:], 1)
    y = (agg_ref[0] + agg_ref[1])[:N] * ni
    mu = jnp.mean(y, axis=0, keepdims=True)
    sub = y - a_ref[:] * mu
    var = jnp.mean(sub * sub, axis=0, keepdims=True)
    h = _leaky(g_ref[:] * sub * lax.rsqrt(var + EPS) + b_ref[:])
    phi = _leaky(jnp.dot(h, pw_ref[:], preferred_element_type=jnp.float32)
                 + pb_ref[:])
    ro = _leaky(jnp.dot(jnp.mean(phi, axis=0, keepdims=True), rw_ref[:],
                        preferred_element_type=jnp.float32) + rb_ref[:])
    ro_ref[:] = _leaky(ro)
    mh_ref[:] = _leaky(jnp.mean(h, axis=0, keepdims=True))


_tc_last = pl.pallas_call(
    _tc_last_body,
    out_shape=[
        jax.ShapeDtypeStruct((1, R), jnp.float32),
        jax.ShapeDtypeStruct((1, H), jnp.float32),
    ],
)


def kernel(node_feats, edge_index, edge_weights, W1, W2, W3,
           a1, g1, b1, a2, g2, b2, a3, g3, b3,
           p1W, p1b, r1W, r1b, p2W, p2b, r2W, r2b, p3W, p3b, r3W, r3b):
    srcs = edge_index[0]
    dsts = edge_index[1]
    row = lambda v: v.reshape(1, -1)

    srcs3 = srcs.reshape(NW, NCHUNK, CH)
    srcs2 = srcs.reshape(NW, EPW)
    dsts3 = dsts.reshape(NW, NCHUNK, CH)
    srcs2 = srcs.reshape(NW, EPW)
    ewb3 = jnp.broadcast_to(edge_weights[:, None], (E, LANES)).reshape(
        NW, NCHUNK, CH * LANES)

    deg = _deg_kernel(srcs3, dsts3)

    hs1, ro0 = _tc_pre(node_feats, W1, deg)
    agg1 = _scatter_kernel(hs1, srcs2, dsts3, ewb3)
    hs2, ro1, mh1 = _tc_mid(agg1, deg, row(a1), row(g1), row(b1),
                            p1W, row(p1b), r1W, row(r1b), W2)
    agg2 = _scatter_kernel(hs2, srcs2, dsts3, ewb3)
    hs3, ro2, mh2 = _tc_mid(agg2, deg, row(a2), row(g2), row(b2),
                            p2W, row(p2b), r2W, row(r2b), W3)
    agg3 = _scatter_kernel(hs3, srcs2, dsts3, ewb3)
    ro3, mh3 = _tc_last(agg3, deg, row(a3), row(g3), row(b3),
                        p3W, row(p3b), r3W, row(r3b))

    return jnp.concatenate([ro0, ro1, mh1, ro2, mh2, ro3, mh3], axis=1)


# speedup vs baseline: 2.6147x; 1.1685x over previous
import functools

import jax
import jax.numpy as jnp
from jax import lax
from jax.experimental import pallas as pl
from jax.experimental.pallas import tpu as pltpu
from jax.experimental.pallas import tpu_sc as plsc

N = 10000
E = 320000
DIN = 128
H = 128
R = 64
EPS = 1e-5

NC = 2
NS = 16
NW = NC * NS
LANES = 16
CH = 80
EPW = E // NW
NCHUNK = EPW // CH
NPAD = 10112
RPT = NPAD // NS
ZR = 8
FV = H // LANES

_MESH = plsc.VectorSubcoreMesh(
    core_axis_name="c", subcore_axis_name="s", num_cores=NC, num_subcores=NS
)


def _leaky(x):
    return jnp.where(x > 0, x, 0.01 * x)


def _deg_body(srcs3, dsts3, deg_out, deg_sp, si0, di0, si1, di1,
              rows_a, rows_b, sem0, sem1, ssem0, ssem1):
    cid = lax.axis_index("c")
    sid = lax.axis_index("s")
    wid = cid * NS + sid

    @pl.loop(0, CH)
    def _fill_zero(r):
        for j in range(FV):
            rows_b[r, pl.ds(j * LANES, LANES)] = jnp.zeros((LANES,), jnp.float32)

    for t in range(RPT // CH):
        r0 = sid * RPT + t * CH
        pltpu.sync_copy(rows_b, deg_sp.at[pl.ds(r0, CH)])
    _tail = RPT - (RPT // CH) * CH
    if _tail:
        pltpu.sync_copy(rows_b.at[pl.ds(0, _tail)],
                        deg_sp.at[pl.ds(sid * RPT + (RPT // CH) * CH, _tail)])

    @pl.loop(0, CH)
    def _fill_ones(r):
        for j in range(FV // 2):
            rows_a[r, pl.ds(j * LANES, LANES)] = jnp.ones((LANES,), jnp.float32)
            rows_b[r, pl.ds((FV // 2 + j) * LANES, LANES)] = jnp.ones(
                (LANES,), jnp.float32)
        for j in range(FV // 2):
            rows_a[r, pl.ds((FV // 2 + j) * LANES, LANES)] = jnp.zeros(
                (LANES,), jnp.float32)
    plsc.subcore_barrier()

    def dstart(k, si, di, sem, ssem, first):
        if not first:
            pltpu.make_async_copy(rows_a, deg_sp.at[si], ssem).wait()
            pltpu.make_async_copy(rows_b, deg_sp.at[di], ssem).wait()
        pltpu.async_copy(srcs3.at[wid, k], si, sem)
        pltpu.async_copy(dsts3.at[wid, k], di, sem)

    def dprocess(si, di, sem, ssem):
        pltpu.make_async_copy(srcs3.at[0, 0], si, sem).wait()
        pltpu.make_async_copy(srcs3.at[0, 0], di, sem).wait()
        pltpu.make_async_copy(rows_a, deg_sp.at[si], ssem).start(add=True)
        pltpu.make_async_copy(rows_b, deg_sp.at[di], ssem).start(add=True)

    dstart(0, si0, di0, sem0, ssem0, True)
    dstart(1, si1, di1, sem1, ssem1, True)

    @pl.loop(0, (NCHUNK + 1) // 2)
    def _pair(g):
        k0 = 2 * g
        dprocess(si0, di0, sem0, ssem0)

        @pl.when(k0 + 2 < NCHUNK)
        def _():
            dstart(k0 + 2, si0, di0, sem0, ssem0, False)

        @pl.when(k0 + 1 < NCHUNK)
        def _():
            dprocess(si1, di1, sem1, ssem1)

            @pl.when(k0 + 3 < NCHUNK)
            def _():
                dstart(k0 + 3, si1, di1, sem1, ssem1, False)

    pltpu.make_async_copy(rows_a, deg_sp.at[si0], ssem0).wait()
    pltpu.make_async_copy(rows_b, deg_sp.at[di0], ssem0).wait()
    pltpu.make_async_copy(rows_a, deg_sp.at[si1], ssem1).wait()
    pltpu.make_async_copy(rows_b, deg_sp.at[di1], ssem1).wait()

    plsc.subcore_barrier()
    r0 = sid * RPT
    pltpu.sync_copy(deg_sp.at[pl.ds(r0, RPT)], deg_out.at[cid, pl.ds(r0, RPT)])


_deg_kernel = pl.kernel(
    _deg_body,
    out_type=jax.ShapeDtypeStruct((NC, NPAD, H), jnp.float32),
    mesh=_MESH,
    scratch_types=[
        pltpu.VMEM_SHARED((NPAD, H), jnp.float32),
        pltpu.VMEM((CH,), jnp.int32),
        pltpu.VMEM((CH,), jnp.int32),
        pltpu.VMEM((CH,), jnp.int32),
        pltpu.VMEM((CH,), jnp.int32),
        pltpu.VMEM((CH, H), jnp.float32),
        pltpu.VMEM((CH, H), jnp.float32),
        pltpu.SemaphoreType.DMA,
        pltpu.SemaphoreType.DMA,
        pltpu.SemaphoreType.DMA,
        pltpu.SemaphoreType.DMA,
    ],
)


def _scatter_body(hs, srcs2, dsts3, ewb3, out, agg_sp,
                  src_all, dst0, dst1, ew0, ew1, rows0, rows1,
                  sem0, sem1, gsem0, gsem1, ssem0, ssem1):
    cid = lax.axis_index("c")
    sid = lax.axis_index("s")
    wid = cid * NS + sid

    @pl.loop(0, CH)
    def _fill_zero(r):
        for j in range(FV):
            rows0[r, pl.ds(j * LANES, LANES)] = jnp.zeros((LANES,), jnp.float32)

    for t in range(RPT // CH):
        r0 = sid * RPT + t * CH
        pltpu.sync_copy(rows0, agg_sp.at[pl.ds(r0, CH)])
    _tail = RPT - (RPT // CH) * CH
    if _tail:
        pltpu.sync_copy(rows0.at[pl.ds(0, _tail)],
                        agg_sp.at[pl.ds(sid * RPT + (RPT // CH) * CH, _tail)])
    pltpu.sync_copy(srcs2.at[wid], src_all)
    plsc.subcore_barrier()

    def start(k, dst_v, ew_v, rows_v, sem, gsem, ssem, first):
        if not first:
            pltpu.make_async_copy(rows_v, agg_sp.at[dst_v], ssem).wait()
        pltpu.async_copy(dsts3.at[wid, k], dst_v, sem)
        pltpu.async_copy(ewb3.at[wid, k], ew_v, sem)
        pltpu.async_copy(hs.at[src_all.at[pl.ds(k * CH, CH)]], rows_v, gsem)

    def finish(k, dst_v, ew_v, rows_v, sem, gsem):
        pltpu.make_async_copy(dsts3.at[0, 0], dst_v, sem).wait()
        pltpu.make_async_copy(ewb3.at[0, 0], ew_v, sem).wait()
        pltpu.make_async_copy(hs.at[src_all.at[pl.ds(k * CH, CH)]],
                              rows_v, gsem).wait()

    def process(dst_v, ew_v, rows_v, ssem):
        @pl.loop(0, CH)
        def _scale(e):
            w = ew_v[pl.ds(e * LANES, LANES)]
            for j in range(FV):
                sl = pl.ds(j * LANES, LANES)
                rows_v[e, sl] = rows_v[e, sl] * w

        pltpu.make_async_copy(rows_v, agg_sp.at[dst_v], ssem).start(add=True)

    start(0, dst0, ew0, rows0, sem0, gsem0, ssem0, True)
    start(1, dst1, ew1, rows1, sem1, gsem1, ssem1, True)

    @pl.loop(0, (NCHUNK + 1) // 2)
    def _pair(g):
        k0 = 2 * g
        finish(k0, dst0, ew0, rows0, sem0, gsem0)
        process(dst0, ew0, rows0, ssem0)

        @pl.when(k0 + 2 < NCHUNK)
        def _():
            start(k0 + 2, dst0, ew0, rows0, sem0, gsem0, ssem0, False)

        @pl.when(k0 + 1 < NCHUNK)
        def _():
            finish(k0 + 1, dst1, ew1, rows1, sem1, gsem1)
            process(dst1, ew1, rows1, ssem1)

            @pl.when(k0 + 3 < NCHUNK)
            def _():
                start(k0 + 3, dst1, ew1, rows1, sem1, gsem1, ssem1, False)

    pltpu.make_async_copy(rows0, agg_sp.at[dst0], ssem0).wait()
    pltpu.make_async_copy(rows1, agg_sp.at[dst1], ssem1).wait()

    plsc.subcore_barrier()
    r0 = sid * RPT
    pltpu.sync_copy(agg_sp.at[pl.ds(r0, RPT)], out.at[cid, pl.ds(r0, RPT)])


_scatter_kernel = pl.kernel(
    _scatter_body,
    out_type=jax.ShapeDtypeStruct((NC, NPAD, H), jnp.float32),
    mesh=_MESH,
    scratch_types=[
        pltpu.VMEM_SHARED((NPAD, H), jnp.float32),
        pltpu.VMEM((EPW,), jnp.int32),
        pltpu.VMEM((CH,), jnp.int32),
        pltpu.VMEM((CH,), jnp.int32),
        pltpu.VMEM((CH * LANES,), jnp.float32),
        pltpu.VMEM((CH * LANES,), jnp.float32),
        pltpu.VMEM((CH, H), jnp.float32),
        pltpu.VMEM((CH, H), jnp.float32),
        pltpu.SemaphoreType.DMA,
        pltpu.SemaphoreType.DMA,
        pltpu.SemaphoreType.DMA,
        pltpu.SemaphoreType.DMA,
        pltpu.SemaphoreType.DMA,
        pltpu.SemaphoreType.DMA,
    ],
)


def _inv_sqrt_deg(degs, half):
    d = (degs[0] + degs[1])[:N, half * (H // 2):half * (H // 2) + (H // 2)]
    d = jnp.max(d, axis=-1, keepdims=True)
    return jnp.where(d > 0, lax.rsqrt(d), 0.0)


def _tc_pre_body(x_ref, w1_ref, deg_ref, hs_ref, ro0_ref):
    x = x_ref[:]
    no = _inv_sqrt_deg(deg_ref[:], 0)
    hs_ref[:] = jnp.dot(x, w1_ref[:], preferred_element_type=jnp.float32) * no
    ro0_ref[:] = _leaky(jnp.mean(x, axis=0, keepdims=True))


_tc_pre = pl.pallas_call(
    _tc_pre_body,
    out_shape=[
        jax.ShapeDtypeStruct((N, H), jnp.float32),
        jax.ShapeDtypeStruct((1, DIN), jnp.float32),
    ],
)


def _tc_mid_body(agg_ref, deg_ref, a_ref, g_ref, b_ref,
                 pw_ref, pb_ref, rw_ref, rb_ref, wn_ref,
                 hsn_ref, ro_ref, mh_ref):
    ni = _inv_sqrt_deg(deg_ref[:], 1)
    y = (agg_ref[0] + agg_ref[1])[:N] * ni
    mu = jnp.mean(y, axis=0, keepdims=True)
    sub = y - a_ref[:] * mu
    var = jnp.mean(sub * sub, axis=0, keepdims=True)
    h = _leaky(g_ref[:] * sub * lax.rsqrt(var + EPS) + b_ref[:])
    phi = _leaky(jnp.dot(h, pw_ref[:], preferred_element_type=jnp.float32)
                 + pb_ref[:])
    ro = _leaky(jnp.dot(jnp.mean(phi, axis=0, keepdims=True), rw_ref[:],
                        preferred_element_type=jnp.float32) + rb_ref[:])
    ro_ref[:] = _leaky(ro)
    mh_ref[:] = _leaky(jnp.mean(h, axis=0, keepdims=True))
    no = _inv_sqrt_deg(deg_ref[:], 0)
    hsn_ref[:] = jnp.dot(h, wn_ref[:], preferred_element_type=jnp.float32) * no


_tc_mid = pl.pallas_call(
    _tc_mid_body,
    out_shape=[
        jax.ShapeDtypeStruct((N, H), jnp.float32),
        jax.ShapeDtypeStruct((1, R), jnp.float32),
        jax.ShapeDtypeStruct((1, H), jnp.float32),
    ],
)


def _tc_last_body(agg_ref, deg_ref, a_ref, g_ref, b_ref,
                  pw_ref, pb_ref, rw_ref, rb_ref,
                  ro_ref, mh_ref):
    ni = _inv_sqrt_deg(deg_ref[:], 1)
    y = (agg_ref[0] + agg_ref[1])[:N] * ni
    mu = jnp.mean(y, axis=0, keepdims=True)
    sub = y - a_ref[:] * mu
    var = jnp.mean(sub * sub, axis=0, keepdims=True)
    h = _leaky(g_ref[:] * sub * lax.rsqrt(var + EPS) + b_ref[:])
    phi = _leaky(jnp.dot(h, pw_ref[:], preferred_element_type=jnp.float32)
                 + pb_ref[:])
    ro = _leaky(jnp.dot(jnp.mean(phi, axis=0, keepdims=True), rw_ref[:],
                        preferred_element_type=jnp.float32) + rb_ref[:])
    ro_ref[:] = _leaky(ro)
    mh_ref[:] = _leaky(jnp.mean(h, axis=0, keepdims=True))


_tc_last = pl.pallas_call(
    _tc_last_body,
    out_shape=[
        jax.ShapeDtypeStruct((1, R), jnp.float32),
        jax.ShapeDtypeStruct((1, H), jnp.float32),
    ],
)


def kernel(node_feats, edge_index, edge_weights, W1, W2, W3,
           a1, g1, b1, a2, g2, b2, a3, g3, b3,
           p1W, p1b, r1W, r1b, p2W, p2b, r2W, r2b, p3W, p3b, r3W, r3b):
    srcs = edge_index[0]
    dsts = edge_index[1]
    row = lambda v: v.reshape(1, -1)

    srcs3 = srcs.reshape(NW, NCHUNK, CH)
    srcs2 = srcs.reshape(NW, EPW)
    dsts3 = dsts.reshape(NW, NCHUNK, CH)
    srcs2 = srcs.reshape(NW, EPW)
    ewb3 = jnp.broadcast_to(edge_weights[:, None], (E, LANES)).reshape(
        NW, NCHUNK, CH * LANES)

    deg = _deg_kernel(srcs3, dsts3)

    hs1, ro0 = _tc_pre(node_feats, W1, deg)
    agg1 = _scatter_kernel(hs1, srcs2, dsts3, ewb3)
    hs2, ro1, mh1 = _tc_mid(agg1, deg, row(a1), row(g1), row(b1),
                            p1W, row(p1b), r1W, row(r1b), W2)
    agg2 = _scatter_kernel(hs2, srcs2, dsts3, ewb3)
    hs3, ro2, mh2 = _tc_mid(agg2, deg, row(a2), row(g2), row(b2),
                            p2W, row(p2b), r2W, row(r2b), W3)
    agg3 = _scatter_kernel(hs3, srcs2, dsts3, ewb3)
    ro3, mh3 = _tc_last(agg3, deg, row(a3), row(g3), row(b3),
                        p3W, row(p3b), r3W, row(r3b))

    return jnp.concatenate([ro0, ro1, mh1, ro2, mh2, ro3, mh3], axis=1)


# speedup vs baseline: 2.7014x; 1.0331x over previous
import functools

import jax
import jax.numpy as jnp
from jax import lax
from jax.experimental import pallas as pl
from jax.experimental.pallas import tpu as pltpu
from jax.experimental.pallas import tpu_sc as plsc

N = 10000
E = 320000
DIN = 128
H = 128
R = 64
EPS = 1e-5

NC = 2
NS = 16
NW = NC * NS
LANES = 16
CH = 80
EPW = E // NW
NCHUNK = EPW // CH
NPAD = 10112
RPT = NPAD // NS
ZR = 8
FV = H // LANES

_MESH = plsc.VectorSubcoreMesh(
    core_axis_name="c", subcore_axis_name="s", num_cores=NC, num_subcores=NS
)


def _leaky(x):
    return jnp.where(x > 0, x, 0.01 * x)


def _deg_body(srcs3, dsts3, deg_out, deg_sp, si0, di0, si1, di1,
              rows_a, rows_b, sem0, sem1, ssem0, ssem1):
    cid = lax.axis_index("c")
    sid = lax.axis_index("s")
    wid = cid * NS + sid

    @pl.loop(0, CH)
    def _fill_zero(r):
        for j in range(FV):
            rows_b[r, pl.ds(j * LANES, LANES)] = jnp.zeros((LANES,), jnp.float32)

    for t in range(RPT // CH):
        r0 = sid * RPT + t * CH
        pltpu.sync_copy(rows_b, deg_sp.at[pl.ds(r0, CH)])
    _tail = RPT - (RPT // CH) * CH
    if _tail:
        pltpu.sync_copy(rows_b.at[pl.ds(0, _tail)],
                        deg_sp.at[pl.ds(sid * RPT + (RPT // CH) * CH, _tail)])

    @pl.loop(0, CH)
    def _fill_ones(r):
        for j in range(FV // 2):
            rows_a[r, pl.ds(j * LANES, LANES)] = jnp.ones((LANES,), jnp.float32)
            rows_b[r, pl.ds((FV // 2 + j) * LANES, LANES)] = jnp.ones(
                (LANES,), jnp.float32)
        for j in range(FV // 2):
            rows_a[r, pl.ds((FV // 2 + j) * LANES, LANES)] = jnp.zeros(
                (LANES,), jnp.float32)
    plsc.subcore_barrier()

    def dstart(k, si, di, sem, ssem, first):
        if not first:
            pltpu.make_async_copy(rows_a, deg_sp.at[si], ssem).wait()
            pltpu.make_async_copy(rows_b, deg_sp.at[di], ssem).wait()
        pltpu.async_copy(srcs3.at[wid, k], si, sem)
        pltpu.async_copy(dsts3.at[wid, k], di, sem)

    def dprocess(si, di, sem, ssem):
        pltpu.make_async_copy(srcs3.at[0, 0], si, sem).wait()
        pltpu.make_async_copy(srcs3.at[0, 0], di, sem).wait()
        pltpu.make_async_copy(rows_a, deg_sp.at[si], ssem).start(add=True)
        pltpu.make_async_copy(rows_b, deg_sp.at[di], ssem).start(add=True)

    dstart(0, si0, di0, sem0, ssem0, True)
    dstart(1, si1, di1, sem1, ssem1, True)

    @pl.loop(0, (NCHUNK + 1) // 2)
    def _pair(g):
        k0 = 2 * g
        dprocess(si0, di0, sem0, ssem0)

        @pl.when(k0 + 2 < NCHUNK)
        def _():
            dstart(k0 + 2, si0, di0, sem0, ssem0, False)

        @pl.when(k0 + 1 < NCHUNK)
        def _():
            dprocess(si1, di1, sem1, ssem1)

            @pl.when(k0 + 3 < NCHUNK)
            def _():
                dstart(k0 + 3, si1, di1, sem1, ssem1, False)

    pltpu.make_async_copy(rows_a, deg_sp.at[si0], ssem0).wait()
    pltpu.make_async_copy(rows_b, deg_sp.at[di0], ssem0).wait()
    pltpu.make_async_copy(rows_a, deg_sp.at[si1], ssem1).wait()
    pltpu.make_async_copy(rows_b, deg_sp.at[di1], ssem1).wait()

    plsc.subcore_barrier()
    r0 = sid * RPT
    pltpu.sync_copy(deg_sp.at[pl.ds(r0, RPT)], deg_out.at[cid, pl.ds(r0, RPT)])


_deg_kernel = pl.kernel(
    _deg_body,
    out_type=jax.ShapeDtypeStruct((NC, NPAD, H), jnp.float32),
    mesh=_MESH,
    scratch_types=[
        pltpu.VMEM_SHARED((NPAD, H), jnp.float32),
        pltpu.VMEM((CH,), jnp.int32),
        pltpu.VMEM((CH,), jnp.int32),
        pltpu.VMEM((CH,), jnp.int32),
        pltpu.VMEM((CH,), jnp.int32),
        pltpu.VMEM((CH, H), jnp.float32),
        pltpu.VMEM((CH, H), jnp.float32),
        pltpu.SemaphoreType.DMA,
        pltpu.SemaphoreType.DMA,
        pltpu.SemaphoreType.DMA,
        pltpu.SemaphoreType.DMA,
    ],
)


def _scatter_body(hs, srcs2, dsts3, ewb3, out, agg_sp,
                  src_all, dst0, dst1, ew0, ew1, rows0, rows1,
                  sem0, sem1, gsem0, gsem1, ssem0, ssem1):
    cid = lax.axis_index("c")
    sid = lax.axis_index("s")
    wid = cid * NS + sid

    @pl.loop(0, CH)
    def _fill_zero(r):
        for j in range(FV):
            rows0[r, pl.ds(j * LANES, LANES)] = jnp.zeros((LANES,), jnp.float32)

    for t in range(RPT // CH):
        r0 = sid * RPT + t * CH
        pltpu.sync_copy(rows0, agg_sp.at[pl.ds(r0, CH)])
    _tail = RPT - (RPT // CH) * CH
    if _tail:
        pltpu.sync_copy(rows0.at[pl.ds(0, _tail)],
                        agg_sp.at[pl.ds(sid * RPT + (RPT // CH) * CH, _tail)])
    pltpu.sync_copy(srcs2.at[wid], src_all)
    plsc.subcore_barrier()

    def start(k, dst_v, ew_v, rows_v, sem, gsem, ssem, first):
        if not first:
            pltpu.make_async_copy(rows_v, agg_sp.at[dst_v], ssem).wait()
        pltpu.async_copy(dsts3.at[wid, k], dst_v, sem)
        pltpu.async_copy(ewb3.at[wid, k], ew_v, sem)
        pltpu.async_copy(hs.at[src_all.at[pl.ds(k * CH, CH)]], rows_v, gsem)

    def finish(k, dst_v, ew_v, rows_v, sem, gsem):
        pltpu.make_async_copy(dsts3.at[0, 0], dst_v, sem).wait()
        pltpu.make_async_copy(ewb3.at[0, 0], ew_v, sem).wait()
        pltpu.make_async_copy(hs.at[src_all.at[pl.ds(k * CH, CH)]],
                              rows_v, gsem).wait()

    def process(dst_v, ew_v, rows_v, ssem):
        @pl.loop(0, CH, unroll=4)
        def _scale(e):
            w = ew_v[pl.ds(e * LANES, LANES)]
            for j in range(FV):
                sl = pl.ds(j * LANES, LANES)
                rows_v[e, sl] = rows_v[e, sl] * w

        pltpu.make_async_copy(rows_v, agg_sp.at[dst_v], ssem).start(add=True)

    start(0, dst0, ew0, rows0, sem0, gsem0, ssem0, True)
    start(1, dst1, ew1, rows1, sem1, gsem1, ssem1, True)

    @pl.loop(0, (NCHUNK + 1) // 2)
    def _pair(g):
        k0 = 2 * g
        finish(k0, dst0, ew0, rows0, sem0, gsem0)
        process(dst0, ew0, rows0, ssem0)

        @pl.when(k0 + 2 < NCHUNK)
        def _():
            start(k0 + 2, dst0, ew0, rows0, sem0, gsem0, ssem0, False)

        @pl.when(k0 + 1 < NCHUNK)
        def _():
            finish(k0 + 1, dst1, ew1, rows1, sem1, gsem1)
            process(dst1, ew1, rows1, ssem1)

            @pl.when(k0 + 3 < NCHUNK)
            def _():
                start(k0 + 3, dst1, ew1, rows1, sem1, gsem1, ssem1, False)

    pltpu.make_async_copy(rows0, agg_sp.at[dst0], ssem0).wait()
    pltpu.make_async_copy(rows1, agg_sp.at[dst1], ssem1).wait()

    plsc.subcore_barrier()
    r0 = sid * RPT
    pltpu.sync_copy(agg_sp.at[pl.ds(r0, RPT)], out.at[cid, pl.ds(r0, RPT)])


_scatter_kernel = pl.kernel(
    _scatter_body,
    out_type=jax.ShapeDtypeStruct((NC, NPAD, H), jnp.float32),
    mesh=_MESH,
    scratch_types=[
        pltpu.VMEM_SHARED((NPAD, H), jnp.float32),
        pltpu.VMEM((EPW,), jnp.int32),
        pltpu.VMEM((CH,), jnp.int32),
        pltpu.VMEM((CH,), jnp.int32),
        pltpu.VMEM((CH * LANES,), jnp.float32),
        pltpu.VMEM((CH * LANES,), jnp.float32),
        pltpu.VMEM((CH, H), jnp.float32),
        pltpu.VMEM((CH, H), jnp.float32),
        pltpu.SemaphoreType.DMA,
        pltpu.SemaphoreType.DMA,
        pltpu.SemaphoreType.DMA,
        pltpu.SemaphoreType.DMA,
        pltpu.SemaphoreType.DMA,
        pltpu.SemaphoreType.DMA,
    ],
)


def _inv_sqrt_deg(degs, half):
    d = (degs[0] + degs[1])[:N, half * (H // 2):half * (H // 2) + (H // 2)]
    d = jnp.max(d, axis=-1, keepdims=True)
    return jnp.where(d > 0, lax.rsqrt(d), 0.0)


def _tc_pre_body(x_ref, w1_ref, deg_ref, hs_ref, ro0_ref):
    x = x_ref[:]
    no = _inv_sqrt_deg(deg_ref[:], 0)
    hs_ref[:] = jnp.dot(x, w1_ref[:], preferred_element_type=jnp.float32) * no
    ro0_ref[:] = _leaky(jnp.mean(x, axis=0, keepdims=True))


_tc_pre = pl.pallas_call(
    _tc_pre_body,
    out_shape=[
        jax.ShapeDtypeStruct((N, H), jnp.float32),
        jax.ShapeDtypeStruct((1, DIN), jnp.float32),
    ],
)


def _tc_mid_body(agg_ref, deg_ref, a_ref, g_ref, b_ref,
                 pw_ref, pb_ref, rw_ref, rb_ref, wn_ref,
                 hsn_ref, ro_ref, mh_ref):
    ni = _inv_sqrt_deg(deg_ref[:], 1)
    y = (agg_ref[0] + agg_ref[1])[:N] * ni
    mu = jnp.mean(y, axis=0, keepdims=True)
    sub = y - a_ref[:] * mu
    var = jnp.mean(sub * sub, axis=0, keepdims=True)
    h = _leaky(g_ref[:] * sub * lax.rsqrt(var + EPS) + b_ref[:])
    phi = _leaky(jnp.dot(h, pw_ref[:], preferred_element_type=jnp.float32)
                 + pb_ref[:])
    ro = _leaky(jnp.dot(jnp.mean(phi, axis=0, keepdims=True), rw_ref[:],
                        preferred_element_type=jnp.float32) + rb_ref[:])
    ro_ref[:] = _leaky(ro)
    mh_ref[:] = _leaky(jnp.mean(h, axis=0, keepdims=True))
    no = _inv_sqrt_deg(deg_ref[:], 0)
    hsn_ref[:] = jnp.dot(h, wn_ref[:], preferred_element_type=jnp.float32) * no


_tc_mid = pl.pallas_call(
    _tc_mid_body,
    out_shape=[
        jax.ShapeDtypeStruct((N, H), jnp.float32),
        jax.ShapeDtypeStruct((1, R), jnp.float32),
        jax.ShapeDtypeStruct((1, H), jnp.float32),
    ],
)


def _tc_last_body(agg_ref, deg_ref, a_ref, g_ref, b_ref,
                  pw_ref, pb_ref, rw_ref, rb_ref,
                  ro_ref, mh_ref):
    ni = _inv_sqrt_deg(deg_ref[:], 1)
    y = (agg_ref[0] + agg_ref[1])[:N] * ni
    mu = jnp.mean(y, axis=0, keepdims=True)
    sub = y - a_ref[:] * mu
    var = jnp.mean(sub * sub, axis=0, keepdims=True)
    h = _leaky(g_ref[:] * sub * lax.rsqrt(var + EPS) + b_ref[:])
    phi = _leaky(jnp.dot(h, pw_ref[:], preferred_element_type=jnp.float32)
                 + pb_ref[:])
    ro = _leaky(jnp.dot(jnp.mean(phi, axis=0, keepdims=True), rw_ref[:],
                        preferred_element_type=jnp.float32) + rb_ref[:])
    ro_ref[:] = _leaky(ro)
    mh_ref[:] = _leaky(jnp.mean(h, axis=0, keepdims=True))


_tc_last = pl.pallas_call(
    _tc_last_body,
    out_shape=[
        jax.ShapeDtypeStruct((1, R), jnp.float32),
        jax.ShapeDtypeStruct((1, H), jnp.float32),
    ],
)


def kernel(node_feats, edge_index, edge_weights, W1, W2, W3,
           a1, g1, b1, a2, g2, b2, a3, g3, b3,
           p1W, p1b, r1W, r1b, p2W, p2b, r2W, r2b, p3W, p3b, r3W, r3b):
    srcs = edge_index[0]
    dsts = edge_index[1]
    row = lambda v: v.reshape(1, -1)

    srcs3 = srcs.reshape(NW, NCHUNK, CH)
    srcs2 = srcs.reshape(NW, EPW)
    dsts3 = dsts.reshape(NW, NCHUNK, CH)
    srcs2 = srcs.reshape(NW, EPW)
    ewb3 = jnp.broadcast_to(edge_weights[:, None], (E, LANES)).reshape(
        NW, NCHUNK, CH * LANES)

    deg = _deg_kernel(srcs3, dsts3)

    hs1, ro0 = _tc_pre(node_feats, W1, deg)
    agg1 = _scatter_kernel(hs1, srcs2, dsts3, ewb3)
    hs2, ro1, mh1 = _tc_mid(agg1, deg, row(a1), row(g1), row(b1),
                            p1W, row(p1b), r1W, row(r1b), W2)
    agg2 = _scatter_kernel(hs2, srcs2, dsts3, ewb3)
    hs3, ro2, mh2 = _tc_mid(agg2, deg, row(a2), row(g2), row(b2),
                            p2W, row(p2b), r2W, row(r2b), W3)
    agg3 = _scatter_kernel(hs3, srcs2, dsts3, ewb3)
    ro3, mh3 = _tc_last(agg3, deg, row(a3), row(g3), row(b3),
                        p3W, row(p3b), r3W, row(r3b))

    return jnp.concatenate([ro0, ro1, mh1, ro2, mh2, ro3, mh3], axis=1)
